# Initial kernel scaffold; baseline (speedup 1.0000x reference)
import jax
import jax.numpy as jnp
from jax.experimental import pallas as pl


def kernel(vid, g_w, g_b, theta_w, theta_b, phi_w, phi_b, W_w, W_b):
    raise NotImplementedError("write your pallas kernel here")


# speedup vs baseline: 1.0020x; 1.0020x over previous
import functools

import jax
import jax.numpy as jnp
from jax.experimental import pallas as pl
from jax.experimental.pallas import tpu as pltpu

Q = 8192
CIN = 64
CD = 16
KS = 100
SCALE = 10.0
RB = 256


def _stage_a_kernel(q_ref, kt_ref, d_ref, kn_s):
    i = pl.program_id(0)

    @pl.when(i == 0)
    def _init():
        kt = kt_ref[...]
        kn_s[...] = jnp.sum(kt * kt, axis=0, keepdims=True)

    qb = q_ref[...]
    qn = jnp.sum(qb * qb, axis=1, keepdims=True)
    qk = jnp.dot(qb, kt_ref[...], preferred_element_type=jnp.float32,
                 precision=jax.lax.Precision.DEFAULT)
    d_ref[...] = -((qn - 2.0 * qk) + kn_s[...])


def _stage_a(q, kT):
    return pl.pallas_call(
        _stage_a_kernel,
        grid=(Q // RB,),
        in_specs=[
            pl.BlockSpec((RB, CD), lambda i: (i, 0)),
            pl.BlockSpec((CD, Q), lambda i: (0, 0)),
        ],
        out_specs=pl.BlockSpec((RB, Q), lambda i: (i, 0)),
        out_shape=jax.ShapeDtypeStruct((Q, Q), jnp.float32),
        scratch_shapes=[pltpu.VMEM((1, Q), jnp.float32)],
    )(q, kT)


def kernel(vid, g_w, g_b, theta_w, theta_b, phi_w, phi_b, W_w, W_b):
    t, c, h, w = vid.shape
    vid_f = vid.transpose(0, 2, 3, 1).reshape(Q, CIN)
    q = jnp.dot(vid_f, g_w.T) + g_b
    k = jnp.dot(vid_f, phi_w.T) + phi_b
    v = jnp.dot(vid_f, theta_w.T) + theta_b

    dists = _stage_a(q, k.T)

    topd, topi = jax.lax.top_k(dists, KS)
    yi = jax.nn.softmax(topd * SCALE, axis=1)
    vg = jnp.take(v, topi, axis=0)
    zi = jnp.einsum('qk,qkd->qd', yi, vg)
    y = zi.reshape(t, h, w, CD).transpose(0, 3, 1, 2)
    y = jnp.einsum('tchw,oc->tohw', y, W_w) + W_b[None, :, None, None]
    y = vid + y
    return (y, topi)
